# Initial kernel scaffold; baseline (speedup 1.0000x reference)
import jax
import jax.numpy as jnp
from jax.experimental import pallas as pl


def kernel(pred_coords, pred_logits, gt_coords, gt_labels, gt_masks):
    raise NotImplementedError("write your pallas kernel here")


# speedup vs baseline: 1016.1294x; 1016.1294x over previous
import functools

import jax
import jax.numpy as jnp
from jax import lax
from jax.experimental import pallas as pl
from jax.experimental.pallas import tpu as pltpu

_COST_POINT = 0.1
_COST_CLASS = 1.0
_K_TOP = 4
_QT = 512
_BIG = 1 << 30


def _matcher_body(coords_ref, probs_ref, gtc_ref, lab_ref, maskf_ref,
                  out_ref, vals_s, idx_s):
    qt = pl.program_id(1)
    nq_tiles = pl.num_programs(1)
    ng = gtc_ref.shape[2]

    @pl.when(qt == 0)
    def _init():
        vals_s[...] = jnp.full(vals_s.shape, jnp.inf, jnp.float32)
        idx_s[...] = jnp.full(idx_s.shape, _BIG, jnp.int32)

    px = coords_ref[0, :, 0:1]
    py = coords_ref[0, :, 1:2]
    gx = gtc_ref[0, 0:1, :]
    gy = gtc_ref[0, 1:2, :]
    dx = px - gx
    dy = py - gy
    dist = jnp.sqrt(dx * dx + dy * dy)

    probs = probs_ref[0, :, :]
    lab = lab_ref[0, :, :]
    b0 = (lab & 1) != 0
    b1 = (lab & 2) != 0
    b2 = lab >= 4
    p = [probs[:, c:c + 1] for c in range(6)]
    q01 = jnp.where(b0, p[1], p[0])
    q23 = jnp.where(b0, p[3], p[2])
    q45 = jnp.where(b0, p[5], p[4])
    q0123 = jnp.where(b1, q23, q01)
    gathered = jnp.where(b2, q45, q0123)

    C = _COST_POINT * dist + _COST_CLASS * (-gathered)
    C = jnp.where(maskf_ref[0, :, :] > 0, C, jnp.inf)

    cur = C
    ridx = lax.broadcasted_iota(jnp.int32, C.shape, 0) + qt * _QT
    tv = []
    ti = []
    for j in range(_K_TOP):
        vmin = jnp.min(cur, axis=0, keepdims=True)
        hit = cur == vmin
        imin = jnp.min(jnp.where(hit, ridx, _BIG), axis=0, keepdims=True)
        tv.append(vmin)
        ti.append(imin)
        if j + 1 < _K_TOP:
            ext = hit & (ridx == imin)
            cur = jnp.where(ext, jnp.inf, cur)
            ridx = jnp.where(ext, _BIG, ridx)

    cv = jnp.concatenate([vals_s[...]] + tv, axis=0)
    ci = jnp.concatenate([idx_s[...]] + ti, axis=0)
    mv = []
    mi = []
    for j in range(_K_TOP):
        vmin = jnp.min(cv, axis=0, keepdims=True)
        hit = cv == vmin
        imin = jnp.min(jnp.where(hit, ci, _BIG), axis=0, keepdims=True)
        mv.append(vmin)
        mi.append(imin)
        ext = hit & (ci == imin)
        cv = jnp.where(ext, jnp.inf, cv)
        ci = jnp.where(ext, _BIG, ci)
    vals_s[...] = jnp.concatenate(mv, axis=0)
    idx_s[...] = jnp.concatenate(mi, axis=0)

    @pl.when(qt == nq_tiles - 1)
    def _finish():
        out_ref[0, :, :] = idx_s[...]


@jax.jit
def kernel(pred_coords, pred_logits, gt_coords, gt_labels, gt_masks):
    bs, nq, _ = pred_coords.shape
    ng = gt_coords.shape[1]
    pred_probs = jax.nn.softmax(pred_logits, axis=-1)
    gtc_t = jnp.swapaxes(gt_coords, 1, 2)
    lab = gt_labels.astype(jnp.int32).reshape(bs, 1, ng)
    maskf = gt_masks.astype(jnp.float32).reshape(bs, 1, ng)
    grid = (bs, nq // _QT)
    out = pl.pallas_call(
        _matcher_body,
        grid=grid,
        in_specs=[
            pl.BlockSpec((1, _QT, 2), lambda b, q: (b, q, 0)),
            pl.BlockSpec((1, _QT, 6), lambda b, q: (b, q, 0)),
            pl.BlockSpec((1, 2, ng), lambda b, q: (b, 0, 0)),
            pl.BlockSpec((1, 1, ng), lambda b, q: (b, 0, 0)),
            pl.BlockSpec((1, 1, ng), lambda b, q: (b, 0, 0)),
        ],
        out_specs=pl.BlockSpec((1, _K_TOP, ng), lambda b, q: (b, 0, 0)),
        out_shape=jax.ShapeDtypeStruct((bs, _K_TOP, ng), jnp.int32),
        scratch_shapes=[
            pltpu.VMEM((_K_TOP, ng), jnp.float32),
            pltpu.VMEM((_K_TOP, ng), jnp.int32),
        ],
        compiler_params=pltpu.CompilerParams(
            dimension_semantics=("arbitrary", "arbitrary")),
    )(pred_coords, pred_probs, gtc_t, lab, maskf)
    return out
